# baseline (speedup 1.0000x reference)
import functools

import jax
import jax.numpy as jnp
from jax import lax
from jax.experimental import pallas as pl
from jax.experimental.pallas import tpu as pltpu
from jax.experimental.pallas import tpu_sc as plsc

N = 10000
E = 320000
D = 128

NC = 2
NS = 16
NW = NC * NS
EPW = E // NW
C = 40
NCH = EPW // C
NP = 10240
NPT = NP // NS

_EPS = 1e-5


def _bn_body(x_ref, g_ref, b_ref, o_ref):
    x = x_ref[...]
    mu = jnp.mean(x, axis=0, keepdims=True)
    xc = x - mu
    var = jnp.mean(xc * xc, axis=0, keepdims=True)
    o_ref[...] = g_ref[...] * xc * lax.rsqrt(var + _EPS) + b_ref[...]


def _edge_lin_body(ea_ref, w_ref, b_ref, o_ref):
    o_ref[...] = (
        jnp.dot(ea_ref[...], w_ref[...], preferred_element_type=jnp.float32)
        + b_ref[...]
    )


def _final_body(h_ref, p_ref, w1_ref, b1_ref, g1_ref, bb1_ref,
                wfc_ref, o_ref):
    z = h_ref[...] + p_ref[0:N, :] + p_ref[NP : NP + N, :]
    t = jnp.tanh(jnp.dot(z, w1_ref[...], preferred_element_type=jnp.float32)
                 + b1_ref[...])
    mu = jnp.mean(t, axis=0, keepdims=True)
    tc = t - mu
    var = jnp.mean(tc * tc, axis=0, keepdims=True)
    tn = g1_ref[...] * tc * lax.rsqrt(var + _EPS) + bb1_ref[...]
    o_ref[...] = jnp.tanh(jnp.dot(tn, wfc_ref[...],
                                  preferred_element_type=jnp.float32))


def _make_sc_body(epw, nch):
  def _sc_aggr_body(h_hbm, e_hbm, src_hbm, dst_hbm, zeros_hbm, out_hbm,
                    sv0, sv1, sv2, sv3, dv0, dv1, dv2, dv3,
                    eb0, eb1, hr0, hr1, ms0, ms1, aggr_sh,
                    se0, se1, sh0, sh1, ss0, ss1,
                    si0, si1, si2, si3, sj0, sj1, sj2, sj3):
    cid = lax.axis_index("c")
    sid = lax.axis_index("s")
    wid = sid * NC + cid
    svs, dvs = (sv0, sv1, sv2, sv3), (dv0, dv1, dv2, dv3)
    ebs, hrs, mss = (eb0, eb1), (hr0, hr1), (ms0, ms1)
    ses, shs, sss = (se0, se1), (sh0, sh1), (ss0, ss1)
    sis, sjs = (si0, si1, si2, si3), (sj0, sj1, sj2, sj3)
    ebase = wid * epw

    def idx_start(j, q):
        pltpu.async_copy(src_hbm.at[pl.ds(ebase + j * C, C)], svs[q], sis[q])
        pltpu.async_copy(dst_hbm.at[pl.ds(ebase + j * C, C)], dvs[q], sjs[q])

    def idx_wait(j, q):
        pltpu.make_async_copy(src_hbm.at[pl.ds(ebase + j * C, C)],
                              svs[q], sis[q]).wait()
        pltpu.make_async_copy(dst_hbm.at[pl.ds(ebase + j * C, C)],
                              dvs[q], sjs[q]).wait()

    def fill_start(j, b, q):
        idx_wait(j, q)
        pltpu.async_copy(e_hbm.at[pl.ds(ebase + j * C, C)], ebs[b], ses[b])
        pltpu.async_copy(h_hbm.at[svs[q]], hrs[b], shs[b])

    def fill_wait(j, b, q):
        pltpu.make_async_copy(e_hbm.at[pl.ds(ebase + j * C, C)],
                              ebs[b], ses[b]).wait()
        pltpu.make_async_copy(h_hbm.at[svs[q]], hrs[b], shs[b]).wait()

    def scatter_wait(b):
        pltpu.make_async_copy(mss[b], aggr_sh.at[dv0], sss[b]).wait()

    def process(j, b, q, ws, idx_next, fill_next):
        fill_wait(j, b, q)
        if ws:
            scatter_wait(b)
        if idx_next:
            idx_start(j + 2, (q + 2) % 4)

        @plsc.parallel_loop(0, C, unroll=8)
        def _row(i):
            for k2 in range(D // 16):
                sl = pl.ds(k2 * 16, 16)
                mss[b][i, sl] = jnp.maximum(ebs[b][i, sl] + hrs[b][i, sl], 0.0)

        pltpu.async_copy(mss[b], aggr_sh.at[dvs[q]], sss[b], add=True)
        if fill_next:
            fill_start(j + 2, b, (q + 2) % 4)

    for q in range(4):
        idx_start(q, q)
    fill_start(0, 0, 0)
    fill_start(1, 1, 1)
    pltpu.sync_copy(zeros_hbm.at[pl.ds(cid * NP + sid * NPT, NPT)],
                    aggr_sh.at[pl.ds(sid * NPT, NPT)])
    plsc.subcore_barrier()

    process(0, 0, 0, False, False, True)
    process(1, 1, 1, False, False, True)

    def quad(j0, carry):
        j = 2 + 4 * j0
        for t in range(4):
            process(j + t, t % 2, (2 + t) % 4, True, True, True)
        return carry

    lax.fori_loop(0, (nch - 5) // 4, quad, 0)
    process(nch - 3, 0, 2, True, True, True)
    process(nch - 2, 1, 3, True, False, False)
    process(nch - 1, 0, 0, True, False, False)
    scatter_wait(0)
    scatter_wait(1)

    plsc.subcore_barrier()
    pltpu.sync_copy(aggr_sh.at[pl.ds(sid * NPT, NPT)],
                    out_hbm.at[pl.ds(cid * NP + sid * NPT, NPT)])

  return _sc_aggr_body


def kernel(x, edge_index, edge_attr, gamma_in, beta_in, W_edge, b_edge,
           W1, b1, gamma1, beta1, Wfc):
    src = edge_index[0].astype(jnp.int32)
    dst = edge_index[1].astype(jnp.int32)
    zeros = jnp.zeros((NC * NP, D), jnp.float32)

    h = pl.pallas_call(
        _bn_body,
        out_shape=jax.ShapeDtypeStruct((N, D), jnp.float32),
    )(x, gamma_in.reshape(1, D), beta_in.reshape(1, D))

    BE = 4000
    E2 = E // 2
    HB = E2 // BE

    def _elin(off):
        return pl.pallas_call(
            _edge_lin_body,
            grid=(HB,),
            in_specs=[
                pl.BlockSpec((BE, D), lambda i: (i + off, 0)),
                pl.BlockSpec((D, D), lambda i: (0, 0)),
                pl.BlockSpec((1, D), lambda i: (0, 0)),
            ],
            out_specs=pl.BlockSpec((BE, D), lambda i: (i, 0)),
            out_shape=jax.ShapeDtypeStruct((E2, D), jnp.float32),
        )(edge_attr, W_edge, b_edge.reshape(1, D))

    mesh = plsc.VectorSubcoreMesh(core_axis_name="c", subcore_axis_name="s",
                                  num_cores=NC, num_subcores=NS)
    sc_body = _make_sc_body(E2 // NW, E2 // NW // C)

    def _sc(e_half, src_half, dst_half, init):
        return pl.kernel(
            sc_body,
            out_type=jax.ShapeDtypeStruct((NC * NP, D), jnp.float32),
            mesh=mesh,
            scratch_types=(
                [pltpu.VMEM((C,), jnp.int32)] * 8
                + [pltpu.VMEM((C, D), jnp.float32)] * 6
                + [pltpu.VMEM_SHARED((NP, D), jnp.float32)]
                + [pltpu.SemaphoreType.DMA] * 14
            ),
        )(h, e_half, src_half, dst_half, init)

    e1 = _elin(0)
    p1 = _sc(e1, src[:E2], dst[:E2], zeros)
    e2 = _elin(HB)
    p2 = _sc(e2, src[E2:], dst[E2:], p1)

    out = pl.pallas_call(
        _final_body,
        out_shape=jax.ShapeDtypeStruct((N, D), jnp.float32),
    )(h, p2, W1, b1.reshape(1, D), gamma1.reshape(1, D),
      beta1.reshape(1, D), Wfc)
    return out


# speedup vs baseline: 1.0146x; 1.0146x over previous
import functools

import jax
import jax.numpy as jnp
from jax import lax
from jax.experimental import pallas as pl
from jax.experimental.pallas import tpu as pltpu
from jax.experimental.pallas import tpu_sc as plsc

N = 10000
E = 320000
D = 128

NC = 2
NS = 16
NW = NC * NS
EPW = E // NW
C = 40
NCH = EPW // C
NP = 10240
NPT = NP // NS

_EPS = 1e-5


def _bn_body(x_ref, g_ref, b_ref, o_ref):
    x = x_ref[...]
    mu = jnp.mean(x, axis=0, keepdims=True)
    xc = x - mu
    var = jnp.mean(xc * xc, axis=0, keepdims=True)
    o_ref[...] = g_ref[...] * xc * lax.rsqrt(var + _EPS) + b_ref[...]


def _edge_lin_body(ea_ref, w_ref, b_ref, o_ref):
    o_ref[...] = (
        jnp.dot(ea_ref[...], w_ref[...], preferred_element_type=jnp.float32)
        + b_ref[...]
    )


def _final_body(h_ref, p_ref, w1_ref, b1_ref, g1_ref, bb1_ref,
                wfc_ref, o_ref):
    z = h_ref[...] + p_ref[0:N, :] + p_ref[NP : NP + N, :]
    t = jnp.tanh(jnp.dot(z, w1_ref[...], preferred_element_type=jnp.float32)
                 + b1_ref[...])
    mu = jnp.mean(t, axis=0, keepdims=True)
    tc = t - mu
    var = jnp.mean(tc * tc, axis=0, keepdims=True)
    tn = g1_ref[...] * tc * lax.rsqrt(var + _EPS) + bb1_ref[...]
    o_ref[...] = jnp.tanh(jnp.dot(tn, wfc_ref[...],
                                  preferred_element_type=jnp.float32))


def _make_sc_body(epw, nch):
  def _sc_aggr_body(h_hbm, e_hbm, src_hbm, dst_hbm, zeros_hbm, out_hbm,
                    sv0, sv1, sv2, sv3, dv0, dv1, dv2, dv3,
                    eb0, eb1, hr0, hr1, ms0, ms1, aggr_sh,
                    se0, se1, sh0, sh1, ss0, ss1,
                    si0, si1, si2, si3, sj0, sj1, sj2, sj3):
    cid = lax.axis_index("c")
    sid = lax.axis_index("s")
    wid = sid * NC + cid
    svs, dvs = (sv0, sv1, sv2, sv3), (dv0, dv1, dv2, dv3)
    ebs, hrs, mss = (eb0, eb1), (hr0, hr1), (ms0, ms1)
    ses, shs, sss = (se0, se1), (sh0, sh1), (ss0, ss1)
    sis, sjs = (si0, si1, si2, si3), (sj0, sj1, sj2, sj3)
    ebase = wid * epw

    def idx_start(j, q):
        pltpu.async_copy(src_hbm.at[pl.ds(ebase + j * C, C)], svs[q], sis[q])
        pltpu.async_copy(dst_hbm.at[pl.ds(ebase + j * C, C)], dvs[q], sjs[q])

    def idx_wait(j, q):
        pltpu.make_async_copy(src_hbm.at[pl.ds(ebase + j * C, C)],
                              svs[q], sis[q]).wait()
        pltpu.make_async_copy(dst_hbm.at[pl.ds(ebase + j * C, C)],
                              dvs[q], sjs[q]).wait()

    def fill_start(j, b, q):
        idx_wait(j, q)
        pltpu.async_copy(e_hbm.at[pl.ds(ebase + j * C, C)], ebs[b], ses[b])
        pltpu.async_copy(h_hbm.at[svs[q]], hrs[b], shs[b])

    def fill_wait(j, b, q):
        pltpu.make_async_copy(e_hbm.at[pl.ds(ebase + j * C, C)],
                              ebs[b], ses[b]).wait()
        pltpu.make_async_copy(h_hbm.at[svs[q]], hrs[b], shs[b]).wait()

    def scatter_wait(b):
        pltpu.make_async_copy(mss[b], aggr_sh.at[dv0], sss[b]).wait()

    def process(j, b, q, ws, idx_next, fill_next):
        fill_wait(j, b, q)
        if ws:
            scatter_wait(b)
        if idx_next:
            idx_start(j + 2, (q + 2) % 4)

        @plsc.parallel_loop(0, C, unroll=4)
        def _row(i):
            for k2 in range(D // 16):
                sl = pl.ds(k2 * 16, 16)
                mss[b][i, sl] = jnp.maximum(ebs[b][i, sl] + hrs[b][i, sl], 0.0)

        pltpu.async_copy(mss[b], aggr_sh.at[dvs[q]], sss[b], add=True)
        if fill_next:
            fill_start(j + 2, b, (q + 2) % 4)

    for q in range(4):
        idx_start(q, q)
    fill_start(0, 0, 0)
    fill_start(1, 1, 1)
    pltpu.sync_copy(zeros_hbm.at[pl.ds(cid * NP + sid * NPT, NPT)],
                    aggr_sh.at[pl.ds(sid * NPT, NPT)])
    plsc.subcore_barrier()

    process(0, 0, 0, False, False, True)
    process(1, 1, 1, False, False, True)

    def quad(j0, carry):
        j = 2 + 4 * j0
        for t in range(4):
            process(j + t, t % 2, (2 + t) % 4, True, True, True)
        return carry

    lax.fori_loop(0, (nch - 5) // 4, quad, 0)
    process(nch - 3, 0, 2, True, True, True)
    process(nch - 2, 1, 3, True, False, False)
    process(nch - 1, 0, 0, True, False, False)
    scatter_wait(0)
    scatter_wait(1)

    plsc.subcore_barrier()
    pltpu.sync_copy(aggr_sh.at[pl.ds(sid * NPT, NPT)],
                    out_hbm.at[pl.ds(cid * NP + sid * NPT, NPT)])

  return _sc_aggr_body


def kernel(x, edge_index, edge_attr, gamma_in, beta_in, W_edge, b_edge,
           W1, b1, gamma1, beta1, Wfc):
    src = edge_index[0].astype(jnp.int32)
    dst = edge_index[1].astype(jnp.int32)
    zeros = jnp.zeros((NC * NP, D), jnp.float32)

    h = pl.pallas_call(
        _bn_body,
        out_shape=jax.ShapeDtypeStruct((N, D), jnp.float32),
    )(x, gamma_in.reshape(1, D), beta_in.reshape(1, D))

    BE = 4000
    E2 = E // 2
    HB = E2 // BE

    def _elin(off):
        return pl.pallas_call(
            _edge_lin_body,
            grid=(HB,),
            in_specs=[
                pl.BlockSpec((BE, D), lambda i: (i + off, 0)),
                pl.BlockSpec((D, D), lambda i: (0, 0)),
                pl.BlockSpec((1, D), lambda i: (0, 0)),
            ],
            out_specs=pl.BlockSpec((BE, D), lambda i: (i, 0)),
            out_shape=jax.ShapeDtypeStruct((E2, D), jnp.float32),
        )(edge_attr, W_edge, b_edge.reshape(1, D))

    mesh = plsc.VectorSubcoreMesh(core_axis_name="c", subcore_axis_name="s",
                                  num_cores=NC, num_subcores=NS)
    sc_body = _make_sc_body(E2 // NW, E2 // NW // C)

    def _sc(e_half, src_half, dst_half, init):
        return pl.kernel(
            sc_body,
            out_type=jax.ShapeDtypeStruct((NC * NP, D), jnp.float32),
            mesh=mesh,
            scratch_types=(
                [pltpu.VMEM((C,), jnp.int32)] * 8
                + [pltpu.VMEM((C, D), jnp.float32)] * 6
                + [pltpu.VMEM_SHARED((NP, D), jnp.float32)]
                + [pltpu.SemaphoreType.DMA] * 14
            ),
        )(h, e_half, src_half, dst_half, init)

    e1 = _elin(0)
    p1 = _sc(e1, src[:E2], dst[:E2], zeros)
    e2 = _elin(HB)
    p2 = _sc(e2, src[E2:], dst[E2:], p1)

    out = pl.pallas_call(
        _final_body,
        out_shape=jax.ShapeDtypeStruct((N, D), jnp.float32),
    )(h, p2, W1, b1.reshape(1, D), gamma1.reshape(1, D),
      beta1.reshape(1, D), Wfc)
    return out


# speedup vs baseline: 1.0315x; 1.0167x over previous
import jax
import jax.numpy as jnp
from jax import lax
from jax.experimental import pallas as pl
from jax.experimental.pallas import tpu as pltpu
from jax.experimental.pallas import tpu_sc as plsc

N = 10000
E = 320000
D = 128

NC = 2
NS = 16
NW = NC * NS
EPW = E // NW
C = 40
NCH = EPW // C
NP = 10240
NPT = NP // NS

_EPS = 1e-5


def _bn_body(x_ref, g_ref, b_ref, o_ref):
    x = x_ref[...]
    mu = jnp.mean(x, axis=0, keepdims=True)
    xc = x - mu
    var = jnp.mean(xc * xc, axis=0, keepdims=True)
    o_ref[...] = g_ref[...] * xc * lax.rsqrt(var + _EPS) + b_ref[...]


def _edge_lin_body(ea_ref, w_ref, b_ref, o_ref):
    o_ref[...] = (
        jnp.dot(ea_ref[...], w_ref[...], preferred_element_type=jnp.float32)
        + b_ref[...]
    )


def _final_body(h_ref, p_ref, w1_ref, b1_ref, g1_ref, bb1_ref,
                wfc_ref, o_ref):
    z = h_ref[...] + p_ref[0:N, :] + p_ref[NP : NP + N, :]
    t = jnp.tanh(jnp.dot(z, w1_ref[...], preferred_element_type=jnp.float32)
                 + b1_ref[...])
    mu = jnp.mean(t, axis=0, keepdims=True)
    tc = t - mu
    var = jnp.mean(tc * tc, axis=0, keepdims=True)
    tn = g1_ref[...] * tc * lax.rsqrt(var + _EPS) + bb1_ref[...]
    o_ref[...] = jnp.tanh(jnp.dot(tn, wfc_ref[...],
                                  preferred_element_type=jnp.float32))


def _make_sc_body(epw, nch):
  def _sc_aggr_body(h_hbm, e_hbm, src_hbm, dst_hbm, zeros_hbm, out_hbm,
                    sv0, sv1, sv2, sv3, dv0, dv1, dv2, dv3,
                    eb0, eb1, hr0, hr1, ms0, ms1, aggr_sh,
                    se0, se1, sh0, sh1, ss0, ss1,
                    si0, si1, si2, si3, sj0, sj1, sj2, sj3):
    cid = lax.axis_index("c")
    sid = lax.axis_index("s")
    wid = sid * NC + cid
    svs, dvs = (sv0, sv1, sv2, sv3), (dv0, dv1, dv2, dv3)
    ebs, hrs, mss = (eb0, eb1), (hr0, hr1), (ms0, ms1)
    ses, shs, sss = (se0, se1), (sh0, sh1), (ss0, ss1)
    sis, sjs = (si0, si1, si2, si3), (sj0, sj1, sj2, sj3)
    ebase = wid * epw

    def idx_start(j, q):
        pltpu.async_copy(src_hbm.at[pl.ds(ebase + j * C, C)], svs[q], sis[q])
        pltpu.async_copy(dst_hbm.at[pl.ds(ebase + j * C, C)], dvs[q], sjs[q])

    def idx_wait(j, q):
        pltpu.make_async_copy(src_hbm.at[pl.ds(ebase + j * C, C)],
                              svs[q], sis[q]).wait()
        pltpu.make_async_copy(dst_hbm.at[pl.ds(ebase + j * C, C)],
                              dvs[q], sjs[q]).wait()

    def fill_start(j, b, q):
        idx_wait(j, q)
        pltpu.async_copy(e_hbm.at[pl.ds(ebase + j * C, C)], ebs[b], ses[b])
        pltpu.async_copy(h_hbm.at[svs[q]], hrs[b], shs[b])

    def fill_wait(j, b, q):
        pltpu.make_async_copy(e_hbm.at[pl.ds(ebase + j * C, C)],
                              ebs[b], ses[b]).wait()
        pltpu.make_async_copy(h_hbm.at[svs[q]], hrs[b], shs[b]).wait()

    def scatter_wait(b):
        pltpu.make_async_copy(mss[b], aggr_sh.at[dv0], sss[b]).wait()

    def process(j, b, q, ws, idx_next, fill_next):
        fill_wait(j, b, q)
        if ws:
            scatter_wait(b)
        if idx_next:
            idx_start(j + 2, (q + 2) % 4)

        @plsc.parallel_loop(0, C, unroll=4)
        def _row(i):
            for k2 in range(D // 16):
                sl = pl.ds(k2 * 16, 16)
                mss[b][i, sl] = jnp.maximum(ebs[b][i, sl] + hrs[b][i, sl], 0.0)

        pltpu.async_copy(mss[b], aggr_sh.at[dvs[q]], sss[b], add=True)
        if fill_next:
            fill_start(j + 2, b, (q + 2) % 4)

    for q in range(4):
        idx_start(q, q)
    fill_start(0, 0, 0)
    fill_start(1, 1, 1)
    pltpu.sync_copy(zeros_hbm.at[pl.ds(cid * NP + sid * NPT, NPT)],
                    aggr_sh.at[pl.ds(sid * NPT, NPT)])
    plsc.subcore_barrier()

    process(0, 0, 0, False, False, True)
    process(1, 1, 1, False, False, True)

    def quad(j0, carry):
        j = 2 + 4 * j0
        for t in range(4):
            process(j + t, t % 2, (2 + t) % 4, True, True, True)
        return carry

    lax.fori_loop(0, (nch - 5) // 4, quad, 0)
    process(nch - 3, 0, 2, True, True, True)
    process(nch - 2, 1, 3, True, False, False)
    process(nch - 1, 0, 0, True, False, False)
    scatter_wait(0)
    scatter_wait(1)

    plsc.subcore_barrier()
    pltpu.sync_copy(aggr_sh.at[pl.ds(sid * NPT, NPT)],
                    out_hbm.at[pl.ds(cid * NP + sid * NPT, NPT)])

  return _sc_aggr_body


def kernel(x, edge_index, edge_attr, gamma_in, beta_in, W_edge, b_edge,
           W1, b1, gamma1, beta1, Wfc):
    src = edge_index[0].astype(jnp.int32)
    dst = edge_index[1].astype(jnp.int32)
    zeros = jnp.zeros((NC * NP, D), jnp.float32)

    h = pl.pallas_call(
        _bn_body,
        out_shape=jax.ShapeDtypeStruct((N, D), jnp.float32),
    )(x, gamma_in.reshape(1, D), beta_in.reshape(1, D))

    BE = 4000
    E2 = E // 2
    HB = E2 // BE

    def _elin(off):
        return pl.pallas_call(
            _edge_lin_body,
            grid=(HB,),
            in_specs=[
                pl.BlockSpec((BE, D), lambda i: (i + off, 0)),
                pl.BlockSpec((D, D), lambda i: (0, 0)),
                pl.BlockSpec((1, D), lambda i: (0, 0)),
            ],
            out_specs=pl.BlockSpec((BE, D), lambda i: (i, 0)),
            out_shape=jax.ShapeDtypeStruct((E2, D), jnp.float32),
        )(edge_attr, W_edge, b_edge.reshape(1, D))

    mesh = plsc.VectorSubcoreMesh(core_axis_name="c", subcore_axis_name="s",
                                  num_cores=NC, num_subcores=NS)
    sc_body = _make_sc_body(E2 // NW, E2 // NW // C)

    def _sc(e_half, src_half, dst_half, init):
        return pl.kernel(
            sc_body,
            out_type=jax.ShapeDtypeStruct((NC * NP, D), jnp.float32),
            mesh=mesh,
            scratch_types=(
                [pltpu.VMEM((C,), jnp.int32)] * 8
                + [pltpu.VMEM((C, D), jnp.float32)] * 6
                + [pltpu.VMEM_SHARED((NP, D), jnp.float32)]
                + [pltpu.SemaphoreType.DMA] * 14
            ),
        )(h, e_half, src_half, dst_half, init)

    e1 = _elin(0)
    p1 = _sc(e1, src[:E2], dst[:E2], zeros)
    e2 = _elin(HB)
    p2 = _sc(e2, src[E2:], dst[E2:], p1)

    out = pl.pallas_call(
        _final_body,
        out_shape=jax.ShapeDtypeStruct((N, D), jnp.float32),
    )(h, p2, W1, b1.reshape(1, D), gamma1.reshape(1, D),
      beta1.reshape(1, D), Wfc)
    return out
